# baseline (speedup 1.0000x reference)
import functools

import jax
import jax.numpy as jnp
from jax import lax
from jax.experimental import pallas as pl
from jax.experimental.pallas import tpu as pltpu
from jax.experimental.pallas import tpu_sc as plsc

_B = 1024
_N = 100000
_K = 128
_NC = 2
_NS = 16
_NW = _NC * _NS
_RPW = _B // _NW
_L = 16


def _selection_body(x_hbm, sel_hbm, out_hbm, sel_v, idx_v, gat_v, sem):
    wid = lax.axis_index("s") * _NC + lax.axis_index("c")
    base_row = wid * _RPW

    pltpu.sync_copy(sel_hbm, sel_v)

    def chunk_body(c, _):
        s = sel_v[pl.ds(c * _L, _L)]

        def row_body(r, _):
            idx_v[pl.ds(r * _K + c * _L, _L)] = s + (base_row + r) * _N
            return 0

        lax.fori_loop(0, _RPW, row_body, 0)
        return 0

    lax.fori_loop(0, _K // _L, chunk_body, 0)

    pltpu.async_copy(x_hbm.at[idx_v], gat_v, sem).wait()

    pltpu.sync_copy(gat_v, out_hbm.at[pl.ds(base_row * _K, _RPW * _K)])


@jax.jit
def _selection_sc(x_flat, sel):
    run = pl.kernel(
        _selection_body,
        out_type=jax.ShapeDtypeStruct((_B * _K,), jnp.float32),
        mesh=plsc.VectorSubcoreMesh(core_axis_name="c", subcore_axis_name="s"),
        scratch_types=[
            pltpu.VMEM((_K,), jnp.int32),
            pltpu.VMEM((_RPW * _K,), jnp.int32),
            pltpu.VMEM((_RPW * _K,), jnp.float32),
            pltpu.SemaphoreType.DMA,
        ],
    )
    return run(x_flat, sel)


def kernel(x, selected):
    x_flat = jnp.reshape(x, (_B * _N,))
    sel = selected.astype(jnp.int32)
    return jnp.reshape(_selection_sc(x_flat, sel), (_B, _K))


# speedup vs baseline: 38.4071x; 38.4071x over previous
import functools

import jax
import jax.numpy as jnp
from jax import lax
from jax.experimental import pallas as pl
from jax.experimental.pallas import tpu as pltpu
from jax.experimental.pallas import tpu_sc as plsc

_B = 1024
_N = 100000
_K = 128
_NC = 2
_NS = 16
_NWU = 16
_RPW = _K // _NWU


def _selection_body(xt_hbm, sel_hbm, out_hbm, idx_v, rows_v, sem):
    wid = lax.axis_index("s") * _NC + lax.axis_index("c")

    @pl.when(wid < _NWU)
    def _():
        base = wid * _RPW
        pltpu.sync_copy(sel_hbm.at[pl.ds(base, _RPW)], idx_v)
        pltpu.async_copy(xt_hbm.at[idx_v], rows_v, sem).wait()
        pltpu.sync_copy(rows_v, out_hbm.at[pl.ds(base, _RPW)])


@jax.jit
def _selection_sc(xt, sel):
    run = pl.kernel(
        _selection_body,
        out_type=jax.ShapeDtypeStruct((_K, _B), jnp.float32),
        mesh=plsc.VectorSubcoreMesh(core_axis_name="c", subcore_axis_name="s"),
        scratch_types=[
            pltpu.VMEM((_RPW,), jnp.int32),
            pltpu.VMEM((_RPW, _B), jnp.float32),
            pltpu.SemaphoreType.DMA,
        ],
    )
    return run(xt, sel)


def kernel(x, selected):
    sel = selected.astype(jnp.int32)
    gathered_t = _selection_sc(x.T, sel)
    return gathered_t.T


# speedup vs baseline: 40.8724x; 1.0642x over previous
import functools

import jax
import jax.numpy as jnp
from jax import lax
from jax.experimental import pallas as pl
from jax.experimental.pallas import tpu as pltpu
from jax.experimental.pallas import tpu_sc as plsc

_B = 1024
_N = 100000
_K = 128
_NC = 2
_NS = 16
_NWU = 16
_RPW = _K // _NWU


def _selection_body(xt_hbm, sel_hbm, out_hbm, idx_v, rows_v, sem):
    wid = lax.axis_index("s")

    base = wid * _RPW
    pltpu.sync_copy(sel_hbm.at[pl.ds(base, _RPW)], idx_v)
    pltpu.async_copy(xt_hbm.at[idx_v], rows_v, sem).wait()
    pltpu.sync_copy(rows_v, out_hbm.at[pl.ds(base, _RPW)])


@jax.jit
def _selection_sc(xt, sel):
    run = pl.kernel(
        _selection_body,
        out_type=jax.ShapeDtypeStruct((_K, _B), jnp.float32),
        mesh=plsc.VectorSubcoreMesh(
            core_axis_name="c", subcore_axis_name="s", num_cores=1
        ),
        scratch_types=[
            pltpu.VMEM((_RPW,), jnp.int32),
            pltpu.VMEM((_RPW, _B), jnp.float32),
            pltpu.SemaphoreType.DMA,
        ],
    )
    return run(xt, sel)


def kernel(x, selected):
    sel = selected.astype(jnp.int32)
    gathered_t = _selection_sc(x.T, sel)
    return gathered_t.T


# speedup vs baseline: 63.2189x; 1.5467x over previous
import functools

import jax
import jax.numpy as jnp
from jax import lax
from jax.experimental import pallas as pl
from jax.experimental.pallas import tpu as pltpu

_B = 1024
_N = 100000
_K = 128
_G = 16
_RPS = _K // _G


def _selection_body(sel_ref, *refs):
    xs = refs[:_RPS]
    out_ref = refs[_RPS]
    t = pl.program_id(0)
    for i in range(_RPS):
        r8 = sel_ref[t * _RPS + i] % 8
        sel_mat = jax.lax.broadcasted_iota(jnp.int32, (8, _B), 0) == r8
        row = jnp.sum(jnp.where(sel_mat, xs[i][...], 0.0), axis=0, keepdims=True)
        out_ref[pl.ds(i, 1), :] = row


@jax.jit
def _selection_tc(xt, sel):
    grid_spec = pltpu.PrefetchScalarGridSpec(
        num_scalar_prefetch=1,
        grid=(_G,),
        in_specs=[
            pl.BlockSpec(
                (8, _B),
                functools.partial(
                    lambda i, t, sel_ref: (sel_ref[t * _RPS + i] // 8, 0), i
                ),
            )
            for i in range(_RPS)
        ],
        out_specs=pl.BlockSpec((_RPS, _B), lambda t, sel_ref: (t, 0)),
    )
    return pl.pallas_call(
        _selection_body,
        grid_spec=grid_spec,
        out_shape=jax.ShapeDtypeStruct((_K, _B), jnp.float32),
    )(sel, *([xt] * _RPS))


def kernel(x, selected):
    sel = selected.astype(jnp.int32)
    gathered_t = _selection_tc(x.T, sel)
    return gathered_t.T


# speedup vs baseline: 65.5825x; 1.0374x over previous
import functools

import jax
import jax.numpy as jnp
from jax import lax
from jax.experimental import pallas as pl
from jax.experimental.pallas import tpu as pltpu

_B = 1024
_N = 100000
_K = 128
_G = 16
_RPS = _K // _G


def _selection_body(sel_ref, *refs):
    xs = refs[:_RPS]
    out_ref = refs[_RPS]
    t = pl.program_id(0)
    for i in range(_RPS):
        r8 = sel_ref[t * _RPS + i] % 8
        out_ref[pl.ds(i, 1), :] = xs[i][pl.ds(r8, 1), :]


@jax.jit
def _selection_tc(xt, sel):
    grid_spec = pltpu.PrefetchScalarGridSpec(
        num_scalar_prefetch=1,
        grid=(_G,),
        in_specs=[
            pl.BlockSpec(
                (8, _B),
                functools.partial(
                    lambda i, t, sel_ref: (sel_ref[t * _RPS + i] // 8, 0), i
                ),
            )
            for i in range(_RPS)
        ],
        out_specs=pl.BlockSpec((_RPS, _B), lambda t, sel_ref: (t, 0)),
    )
    return pl.pallas_call(
        _selection_body,
        grid_spec=grid_spec,
        out_shape=jax.ShapeDtypeStruct((_K, _B), jnp.float32),
    )(sel, *([xt] * _RPS))


def kernel(x, selected):
    sel = selected.astype(jnp.int32)
    gathered_t = _selection_tc(x.T, sel)
    return gathered_t.T


# speedup vs baseline: 132.2058x; 2.0159x over previous
import functools

import jax
import jax.numpy as jnp
from jax import lax
from jax.experimental import pallas as pl
from jax.experimental.pallas import tpu as pltpu

_B = 1024
_N = 100000
_K = 128
_G = 1
_RPS = _K // _G


def _selection_body(sel_ref, *refs):
    xs = refs[:_RPS]
    out_ref = refs[_RPS]
    t = pl.program_id(0)
    for i in range(_RPS):
        r8 = sel_ref[t * _RPS + i] % 8
        out_ref[pl.ds(i, 1), :] = xs[i][pl.ds(r8, 1), :]


@jax.jit
def _selection_tc(xt, sel):
    grid_spec = pltpu.PrefetchScalarGridSpec(
        num_scalar_prefetch=1,
        grid=(_G,),
        in_specs=[
            pl.BlockSpec(
                (8, _B),
                functools.partial(
                    lambda i, t, sel_ref: (sel_ref[t * _RPS + i] // 8, 0), i
                ),
            )
            for i in range(_RPS)
        ],
        out_specs=pl.BlockSpec((_RPS, _B), lambda t, sel_ref: (t, 0)),
    )
    return pl.pallas_call(
        _selection_body,
        grid_spec=grid_spec,
        out_shape=jax.ShapeDtypeStruct((_K, _B), jnp.float32),
    )(sel, *([xt] * _RPS))


def kernel(x, selected):
    sel = selected.astype(jnp.int32)
    gathered_t = _selection_tc(x.T, sel)
    return gathered_t.T
